# Initial kernel scaffold; baseline (speedup 1.0000x reference)
import jax
import jax.numpy as jnp
from jax.experimental import pallas as pl


def kernel(distance, FEATURE):
    raise NotImplementedError("write your pallas kernel here")


# speedup vs baseline: 1.4905x; 1.4905x over previous
import functools

import jax
import jax.numpy as jnp
import numpy as np
from jax import lax
from jax.experimental import pallas as pl
from jax.experimental.pallas import tpu as pltpu
from jax.experimental.pallas import tpu_sc as plsc

_VMIN, _VMAX, _BINS = 0.0, 8.0, 64
_GAMMAS = (100.0, 10.0, 1.0)
_D = 128
_E = 320000
_W = 3 * _D + 3 * _BINS

_NC, _NS, _L = 2, 16, 16
_NW = _NC * _NS
_PER_W = _E // _NW
_CHUNK = 80
_N_CHUNKS = _PER_W // _CHUNK

_CENTERS = np.linspace(_VMIN, _VMAX, _BINS)


def _sc_kernel(i0_hbm, i1_hbm, i2_hbm, d_hbm, feat_hbm, out_hbm,
               idx0_v, idx1_v, idx2_v, d_v, g0_v, g1_v, g2_v, rbf_v,
               sem0, sem1, sem2):
    wid = lax.axis_index("s") * _NC + lax.axis_index("c")
    iota = lax.iota(jnp.int32, _L)

    def chunk_body(g, carry):
        base = wid * _PER_W + g * _CHUNK

        pltpu.sync_copy(i0_hbm.at[pl.ds(base, _CHUNK)], idx0_v)
        pltpu.sync_copy(i1_hbm.at[pl.ds(base, _CHUNK)], idx1_v)
        pltpu.sync_copy(i2_hbm.at[pl.ds(base, _CHUNK)], idx2_v)
        pltpu.sync_copy(d_hbm.at[pl.ds(base, _CHUNK)], d_v)

        cp0 = pltpu.async_copy(feat_hbm.at[idx0_v], g0_v, sem0)
        cp1 = pltpu.async_copy(feat_hbm.at[idx1_v], g1_v, sem1)
        cp2 = pltpu.async_copy(feat_hbm.at[idx2_v], g2_v, sem2)

        def grp_body(i, c2):
            d16 = d_v[pl.ds(i * _L, _L)]
            rows = i * _L + iota
            for c_i, c in enumerate(_CENTERS):
                t = d16 - jnp.float32(c)
                t2 = t * t
                for g_i, gam in enumerate(_GAMMAS):
                    v = jnp.exp(t2 * jnp.float32(-gam))
                    cols = jnp.full((_L,), g_i * _BINS + c_i, jnp.int32)
                    plsc.store_scatter(rbf_v, [rows, cols], v)
            return c2

        lax.fori_loop(0, _CHUNK // _L, grp_body, 0)

        cp0.wait()
        cp1.wait()
        cp2.wait()

        rows_out = out_hbm.at[pl.ds(base, _CHUNK)]
        pltpu.sync_copy(g0_v, rows_out.at[:, pl.ds(0, _D)])
        pltpu.sync_copy(g1_v, rows_out.at[:, pl.ds(_D, _D)])
        pltpu.sync_copy(g2_v, rows_out.at[:, pl.ds(2 * _D, _D)])
        pltpu.sync_copy(rbf_v, rows_out.at[:, pl.ds(3 * _D, 3 * _BINS)])
        return carry

    lax.fori_loop(0, _N_CHUNKS, chunk_body, 0)


@jax.jit
def _rbf_triangle(i0, i1, i2, d, FEATURE):
    mesh = plsc.VectorSubcoreMesh(
        core_axis_name="c", subcore_axis_name="s",
        num_cores=_NC, num_subcores=_NS)
    f = pl.kernel(
        _sc_kernel,
        out_type=jax.ShapeDtypeStruct((_E, _W), jnp.float32),
        mesh=mesh,
        compiler_params=pltpu.CompilerParams(
            use_tc_tiling_on_sc=False, needs_layout_passes=False),
        scratch_types=[
            pltpu.VMEM((_CHUNK,), jnp.int32),
            pltpu.VMEM((_CHUNK,), jnp.int32),
            pltpu.VMEM((_CHUNK,), jnp.int32),
            pltpu.VMEM((_CHUNK,), jnp.float32),
            pltpu.VMEM((_CHUNK, _D), jnp.float32),
            pltpu.VMEM((_CHUNK, _D), jnp.float32),
            pltpu.VMEM((_CHUNK, _D), jnp.float32),
            pltpu.VMEM((_CHUNK, 3 * _BINS), jnp.float32),
            pltpu.SemaphoreType.DMA,
            pltpu.SemaphoreType.DMA,
            pltpu.SemaphoreType.DMA,
        ],
    )
    return f(i0, i1, i2, d, FEATURE)


def kernel(distance, FEATURE):
    idx = distance[:, :3].astype(jnp.int32)
    d = distance[:, 3]
    return _rbf_triangle(idx[:, 0], idx[:, 1], idx[:, 2], d, FEATURE)


# speedup vs baseline: 1.6297x; 1.0934x over previous
import functools

import jax
import jax.numpy as jnp
import numpy as np
from jax import lax
from jax.experimental import pallas as pl
from jax.experimental.pallas import tpu as pltpu
from jax.experimental.pallas import tpu_sc as plsc

_VMIN, _VMAX, _BINS = 0.0, 8.0, 64
_GAMMAS = (100.0, 10.0, 1.0)
_D = 128
_E = 320000
_W = 3 * _D + 3 * _BINS

_NC, _NS, _L = 2, 16, 16
_NW = _NC * _NS
_PER_W = _E // _NW
_CHUNK = 80
_N_CHUNKS = _PER_W // _CHUNK
_N_PAIRS = (_N_CHUNKS - 1) // 2

_CENTERS = np.linspace(_VMIN, _VMAX, _BINS)


def _sc_kernel(i0_hbm, i1_hbm, i2_hbm, d_hbm, feat_hbm, out_hbm,
               idx_v, d_v, g_v, rbf_v,
               si0, si1, sg0, sg1, sw0, sw1):
    sem_i = (si0, si1)
    sem_g = (sg0, sg1)
    sem_w = (sw0, sw1)
    i_hbm = (i0_hbm, i1_hbm, i2_hbm)
    wid = lax.axis_index("s") * _NC + lax.axis_index("c")
    w0 = wid * _PER_W
    iota = lax.iota(jnp.int32, _L)

    def stage_idx(c, s):
        base = w0 + c * _CHUNK
        for j in range(3):
            pltpu.async_copy(i_hbm[j].at[pl.ds(base, _CHUNK)],
                             idx_v.at[s, j], sem_i[s])
        pltpu.async_copy(d_hbm.at[pl.ds(base, _CHUNK)], d_v.at[s], sem_i[s])

    def wait_idx(c, s):
        base = w0 + c * _CHUNK
        for j in range(3):
            pltpu.make_async_copy(i_hbm[j].at[pl.ds(base, _CHUNK)],
                                  idx_v.at[s, j], sem_i[s]).wait()
        pltpu.make_async_copy(d_hbm.at[pl.ds(base, _CHUNK)],
                              d_v.at[s], sem_i[s]).wait()

    def start_gathers(s):
        for j in range(3):
            pltpu.async_copy(feat_hbm.at[idx_v.at[s, j]], g_v.at[s, j],
                             sem_g[s])

    def wait_gathers(s):
        for j in range(3):
            pltpu.make_async_copy(feat_hbm.at[idx_v.at[s, j]], g_v.at[s, j],
                                  sem_g[s]).wait()

    def issue_writes(c, s):
        base = w0 + c * _CHUNK
        rows = out_hbm.at[pl.ds(base, _CHUNK)]
        for j in range(3):
            pltpu.async_copy(g_v.at[s, j], rows.at[:, pl.ds(j * _D, _D)],
                             sem_w[s])
        pltpu.async_copy(rbf_v.at[s], rows.at[:, pl.ds(3 * _D, 3 * _BINS)],
                         sem_w[s])

    def wait_writes(c, s):
        base = w0 + c * _CHUNK
        rows = out_hbm.at[pl.ds(base, _CHUNK)]
        for j in range(3):
            pltpu.make_async_copy(g_v.at[s, j], rows.at[:, pl.ds(j * _D, _D)],
                                  sem_w[s]).wait()
        pltpu.make_async_copy(rbf_v.at[s],
                              rows.at[:, pl.ds(3 * _D, 3 * _BINS)],
                              sem_w[s]).wait()

    def compute_rbf(s):
        dv = d_v.at[s]
        rv = rbf_v.at[s]

        def grp_body(i, carry):
            d16 = dv[pl.ds(i * _L, _L)]
            rows = i * _L + iota
            for c_i, c in enumerate(_CENTERS):
                t = d16 - jnp.float32(c)
                t2 = t * t
                for g_i, gam in enumerate(_GAMMAS):
                    v = jnp.exp(t2 * jnp.float32(-gam))
                    cols = jnp.full((_L,), g_i * _BINS + c_i, jnp.int32)
                    plsc.store_scatter(rv, [rows, cols], v)
            return carry

        lax.fori_loop(0, _CHUNK // _L, grp_body, 0)

    stage_idx(0, 0)
    wait_idx(0, 0)
    start_gathers(0)
    stage_idx(1, 1)

    def pair_body(k, carry):
        for b in range(2):
            cur, nxt = b, 1 - b
            c = 2 * k + b
            if b == 0:
                @pl.when(k > 0)
                def _():
                    wait_writes(c - 1, nxt)
            else:
                wait_writes(c - 1, nxt)
            wait_idx(c + 1, nxt)
            start_gathers(nxt)
            compute_rbf(cur)
            wait_gathers(cur)
            if b == 0:
                stage_idx(c + 2, cur)
            else:
                @pl.when(k < _N_PAIRS - 1)
                def _():
                    stage_idx(c + 2, cur)
            issue_writes(c, cur)
        return carry

    lax.fori_loop(0, _N_PAIRS, pair_body, 0)

    last = _N_CHUNKS - 1
    compute_rbf(0)
    wait_gathers(0)
    issue_writes(last, 0)
    wait_writes(last - 1, 1)
    wait_writes(last, 0)


@jax.jit
def _rbf_triangle(i0, i1, i2, d, FEATURE):
    mesh = plsc.VectorSubcoreMesh(
        core_axis_name="c", subcore_axis_name="s",
        num_cores=_NC, num_subcores=_NS)
    f = pl.kernel(
        _sc_kernel,
        out_type=jax.ShapeDtypeStruct((_E, _W), jnp.float32),
        mesh=mesh,
        compiler_params=pltpu.CompilerParams(
            use_tc_tiling_on_sc=False, needs_layout_passes=False),
        scratch_types=[
            pltpu.VMEM((2, 3, _CHUNK), jnp.int32),
            pltpu.VMEM((2, _CHUNK), jnp.float32),
            pltpu.VMEM((2, 3, _CHUNK, _D), jnp.float32),
            pltpu.VMEM((2, _CHUNK, 3 * _BINS), jnp.float32),
            pltpu.SemaphoreType.DMA,
            pltpu.SemaphoreType.DMA,
            pltpu.SemaphoreType.DMA,
            pltpu.SemaphoreType.DMA,
            pltpu.SemaphoreType.DMA,
            pltpu.SemaphoreType.DMA,
        ],
    )
    return f(i0, i1, i2, d, FEATURE)


def kernel(distance, FEATURE):
    idx = distance[:, :3].astype(jnp.int32)
    d = distance[:, 3]
    return _rbf_triangle(idx[:, 0], idx[:, 1], idx[:, 2], d, FEATURE)


# speedup vs baseline: 2.2936x; 1.4074x over previous
import functools

import jax
import jax.numpy as jnp
import numpy as np
from jax import lax
from jax.experimental import pallas as pl
from jax.experimental.pallas import tpu as pltpu
from jax.experimental.pallas import tpu_sc as plsc

_VMIN, _VMAX, _BINS = 0.0, 8.0, 64
_GAMMAS = (100.0, 10.0, 1.0)
_D = 128
_E = 320000
_W = 3 * _D + 3 * _BINS

_NC, _NS, _L = 2, 16, 16
_NW = _NC * _NS
_PER_W = _E // _NW
_CHUNK = 80
_N_CHUNKS = _PER_W // _CHUNK
_N_PAIRS = (_N_CHUNKS - 1) // 2

_CENTERS = np.linspace(_VMIN, _VMAX, _BINS)


def _sc_kernel(i0_hbm, i1_hbm, i2_hbm, feat_hbm, out_hbm,
               idx_v, g_v, si0, si1, sg0, sg1, sw0, sw1):
    sem_i = (si0, si1)
    sem_g = (sg0, sg1)
    sem_w = (sw0, sw1)
    i_hbm = (i0_hbm, i1_hbm, i2_hbm)
    wid = lax.axis_index("s") * _NC + lax.axis_index("c")
    w0 = wid * _PER_W

    def stage_idx(c, s):
        base = w0 + c * _CHUNK
        for j in range(3):
            pltpu.async_copy(i_hbm[j].at[pl.ds(base, _CHUNK)],
                             idx_v.at[s, j], sem_i[s])

    def wait_idx(c, s):
        base = w0 + c * _CHUNK
        for j in range(3):
            pltpu.make_async_copy(i_hbm[j].at[pl.ds(base, _CHUNK)],
                                  idx_v.at[s, j], sem_i[s]).wait()

    def start_gathers(s):
        for j in range(3):
            pltpu.async_copy(feat_hbm.at[idx_v.at[s, j]], g_v.at[s, j],
                             sem_g[s])

    def wait_gathers(s):
        for j in range(3):
            pltpu.make_async_copy(feat_hbm.at[idx_v.at[s, j]], g_v.at[s, j],
                                  sem_g[s]).wait()

    def issue_writes(c, s):
        base = w0 + c * _CHUNK
        rows = out_hbm.at[pl.ds(base, _CHUNK)]
        for j in range(3):
            pltpu.async_copy(g_v.at[s, j], rows.at[:, pl.ds(j * _D, _D)],
                             sem_w[s])

    def wait_writes(c, s):
        base = w0 + c * _CHUNK
        rows = out_hbm.at[pl.ds(base, _CHUNK)]
        for j in range(3):
            pltpu.make_async_copy(g_v.at[s, j], rows.at[:, pl.ds(j * _D, _D)],
                                  sem_w[s]).wait()

    stage_idx(0, 0)
    wait_idx(0, 0)
    start_gathers(0)
    stage_idx(1, 1)

    def pair_body(k, carry):
        for b in range(2):
            cur, nxt = b, 1 - b
            c = 2 * k + b
            if b == 0:
                @pl.when(k > 0)
                def _():
                    wait_writes(c - 1, nxt)
            else:
                wait_writes(c - 1, nxt)
            wait_idx(c + 1, nxt)
            start_gathers(nxt)
            wait_gathers(cur)
            if b == 0:
                stage_idx(c + 2, cur)
            else:
                @pl.when(k < _N_PAIRS - 1)
                def _():
                    stage_idx(c + 2, cur)
            issue_writes(c, cur)
        return carry

    lax.fori_loop(0, _N_PAIRS, pair_body, 0)

    last = _N_CHUNKS - 1
    wait_gathers(0)
    issue_writes(last, 0)
    wait_writes(last - 1, 1)
    wait_writes(last, 0)


def _sc_gather(i0, i1, i2, FEATURE):
    mesh = plsc.VectorSubcoreMesh(
        core_axis_name="c", subcore_axis_name="s",
        num_cores=_NC, num_subcores=_NS)
    f = pl.kernel(
        _sc_kernel,
        out_type=jax.ShapeDtypeStruct((_E, _W), jnp.float32),
        mesh=mesh,
        scratch_types=[
            pltpu.VMEM((2, 3, _CHUNK), jnp.int32),
            pltpu.VMEM((2, 3, _CHUNK, _D), jnp.float32),
            pltpu.SemaphoreType.DMA,
            pltpu.SemaphoreType.DMA,
            pltpu.SemaphoreType.DMA,
            pltpu.SemaphoreType.DMA,
            pltpu.SemaphoreType.DMA,
            pltpu.SemaphoreType.DMA,
        ],
    )
    return f(i0, i1, i2, FEATURE)


_TC_BE = 512
_RBF_W = 3 * _BINS

def _tc_rbf_kernel(d_ref, sc_ref, out_ref):
    del sc_ref
    j = pl.program_id(1)
    r = lax.broadcasted_iota(jnp.int32, (1, _D), 1) + j * _D
    cen = (r % _BINS).astype(jnp.float32) * ((_VMAX - _VMIN) / (_BINS - 1))
    band = r // _BINS
    gam = jnp.where(band == 0, _GAMMAS[0],
                    jnp.where(band == 1, _GAMMAS[1], _GAMMAS[2]))
    d = d_ref[:, :]
    t = d - cen
    out_ref[:, :] = jnp.exp(t * t * (-gam))


def _tc_rbf(d2, sc_out):
    return pl.pallas_call(
        _tc_rbf_kernel,
        out_shape=jax.ShapeDtypeStruct((_E, _W), jnp.float32),
        grid=(_E // _TC_BE, 2),
        in_specs=[
            pl.BlockSpec((_TC_BE, 1), lambda i, j: (i, 0)),
            pl.BlockSpec(memory_space=pl.ANY),
        ],
        out_specs=pl.BlockSpec((_TC_BE, _D), lambda i, j: (i, 3 + j)),
        input_output_aliases={1: 0},
    )(d2, sc_out)


@jax.jit
def _rbf_triangle(distance, FEATURE):
    idx = distance[:, :3].astype(jnp.int32)
    d2 = distance[:, 3:4]
    sc_out = _sc_gather(idx[:, 0], idx[:, 1], idx[:, 2], FEATURE)
    return _tc_rbf(d2, sc_out)


def kernel(distance, FEATURE):
    return _rbf_triangle(distance, FEATURE)


# speedup vs baseline: 2.6923x; 1.1739x over previous
import functools

import jax
import jax.numpy as jnp
import numpy as np
from jax import lax
from jax.experimental import pallas as pl
from jax.experimental.pallas import tpu as pltpu
from jax.experimental.pallas import tpu_sc as plsc

_VMIN, _VMAX, _BINS = 0.0, 8.0, 64
_GAMMAS = (100.0, 10.0, 1.0)
_D = 128
_E = 320000
_W = 3 * _D + 3 * _BINS

_NC, _NS, _L = 2, 16, 16
_NW = _NC * _NS
_PER_W = _E // _NW
_CHUNK = 80
_N_CHUNKS = _PER_W // _CHUNK
_N_PAIRS = (_N_CHUNKS - 1) // 2

_CENTERS = np.linspace(_VMIN, _VMAX, _BINS)


def _sc_kernel(i0_hbm, i1_hbm, i2_hbm, feat_hbm, out_hbm,
               idx_v, g_v, si0, si1, sg0, sg1, sw0, sw1):
    sem_i = (si0, si1)
    sem_g = (sg0, sg1)
    sem_w = (sw0, sw1)
    i_hbm = (i0_hbm, i1_hbm, i2_hbm)
    wid = lax.axis_index("s") * _NC + lax.axis_index("c")
    w0 = wid * _PER_W

    def stage_idx(c, s):
        base = w0 + c * _CHUNK
        for j in range(3):
            pltpu.async_copy(i_hbm[j].at[pl.ds(base, _CHUNK)],
                             idx_v.at[s, j], sem_i[s])

    def wait_idx(c, s):
        base = w0 + c * _CHUNK
        for j in range(3):
            pltpu.make_async_copy(i_hbm[j].at[pl.ds(base, _CHUNK)],
                                  idx_v.at[s, j], sem_i[s]).wait()

    def start_gathers(s):
        for j in range(3):
            pltpu.async_copy(feat_hbm.at[idx_v.at[s, j]], g_v.at[s, j],
                             sem_g[s])

    def wait_gathers(s):
        for j in range(3):
            pltpu.make_async_copy(feat_hbm.at[idx_v.at[s, j]], g_v.at[s, j],
                                  sem_g[s]).wait()

    def issue_writes(c, s):
        base = w0 + c * _CHUNK
        rows = out_hbm.at[pl.ds(base, _CHUNK)]
        for j in range(3):
            pltpu.async_copy(g_v.at[s, j], rows.at[:, pl.ds(j * _D, _D)],
                             sem_w[s])

    def wait_writes(c, s):
        base = w0 + c * _CHUNK
        rows = out_hbm.at[pl.ds(base, _CHUNK)]
        for j in range(3):
            pltpu.make_async_copy(g_v.at[s, j], rows.at[:, pl.ds(j * _D, _D)],
                                  sem_w[s]).wait()

    stage_idx(0, 0)
    wait_idx(0, 0)
    start_gathers(0)
    stage_idx(1, 1)

    def pair_body(k, carry):
        for b in range(2):
            cur, nxt = b, 1 - b
            c = 2 * k + b
            if b == 0:
                @pl.when(k > 0)
                def _():
                    wait_writes(c - 1, nxt)
            else:
                wait_writes(c - 1, nxt)
            wait_idx(c + 1, nxt)
            start_gathers(nxt)
            wait_gathers(cur)
            if b == 0:
                stage_idx(c + 2, cur)
            else:
                @pl.when(k < _N_PAIRS - 1)
                def _():
                    stage_idx(c + 2, cur)
            issue_writes(c, cur)
        return carry

    lax.fori_loop(0, _N_PAIRS, pair_body, 0)

    last = _N_CHUNKS - 1
    wait_gathers(0)
    issue_writes(last, 0)
    wait_writes(last - 1, 1)
    wait_writes(last, 0)


def _sc_gather(i0, i1, i2, FEATURE):
    mesh = plsc.VectorSubcoreMesh(
        core_axis_name="c", subcore_axis_name="s",
        num_cores=_NC, num_subcores=_NS)
    f = pl.kernel(
        _sc_kernel,
        out_type=jax.ShapeDtypeStruct((_E, _W), jnp.float32),
        mesh=mesh,
        scratch_types=[
            pltpu.VMEM((2, 3, _CHUNK), jnp.int32),
            pltpu.VMEM((2, 3, _CHUNK, _D), jnp.float32),
            pltpu.SemaphoreType.DMA,
            pltpu.SemaphoreType.DMA,
            pltpu.SemaphoreType.DMA,
            pltpu.SemaphoreType.DMA,
            pltpu.SemaphoreType.DMA,
            pltpu.SemaphoreType.DMA,
        ],
    )
    return f(i0, i1, i2, FEATURE)


_TC_BE = 512
_RBF_W = 3 * _BINS

def _tc_rbf_kernel(d_ref, sc_ref, out_ref):
    del sc_ref
    j = pl.program_id(1)
    r = lax.broadcasted_iota(jnp.int32, (1, _D), 1) + j * _D
    cen = (r % _BINS).astype(jnp.float32) * ((_VMAX - _VMIN) / (_BINS - 1))
    band = r // _BINS
    gam = jnp.where(band == 0, _GAMMAS[0],
                    jnp.where(band == 1, _GAMMAS[1], _GAMMAS[2]))
    d = d_ref[...].reshape(_TC_BE, 1)
    t = d - cen
    out_ref[:, :] = jnp.exp(t * t * (-gam))


def _tc_rbf(d, sc_out):
    return pl.pallas_call(
        _tc_rbf_kernel,
        out_shape=jax.ShapeDtypeStruct((_E, _W), jnp.float32),
        grid=(_E // _TC_BE, 2),
        in_specs=[
            pl.BlockSpec((_TC_BE,), lambda i, j: (i,)),
            pl.BlockSpec(memory_space=pl.ANY),
        ],
        out_specs=pl.BlockSpec((_TC_BE, _D), lambda i, j: (i, 3 + j)),
        input_output_aliases={1: 0},
    )(d, sc_out)


@jax.jit
def _rbf_triangle(distance, FEATURE):
    idx = distance[:, :3].astype(jnp.int32)
    d = distance[:, 3]
    sc_out = _sc_gather(idx[:, 0], idx[:, 1], idx[:, 2], FEATURE)
    return _tc_rbf(d, sc_out)


def kernel(distance, FEATURE):
    return _rbf_triangle(distance, FEATURE)


# speedup vs baseline: 2.9581x; 1.0987x over previous
import functools

import jax
import jax.numpy as jnp
import numpy as np
from jax import lax
from jax.experimental import pallas as pl
from jax.experimental.pallas import tpu as pltpu
from jax.experimental.pallas import tpu_sc as plsc

_VMIN, _VMAX, _BINS = 0.0, 8.0, 64
_GAMMAS = (100.0, 10.0, 1.0)
_D = 128
_E = 320000
_GW = 3 * _D
_W = _GW + 3 * _BINS

_NC, _NS, _L = 2, 16, 16
_NW = _NC * _NS
_PER_W = _E // _NW
_CHUNK = 80
_N_CHUNKS = _PER_W // _CHUNK
_N_PAIRS = (_N_CHUNKS - 1) // 2


def _sc_kernel(i0_hbm, i1_hbm, i2_hbm, feat_hbm, out_hbm,
               idx_v, g_v, si0, si1, sg0, sg1, sw0, sw1):
    sem_i = (si0, si1)
    sem_g = (sg0, sg1)
    sem_w = (sw0, sw1)
    i_hbm = (i0_hbm, i1_hbm, i2_hbm)
    wid = lax.axis_index("s") * _NC + lax.axis_index("c")
    w0 = wid * _PER_W

    def stage_idx(c, s):
        base = w0 + c * _CHUNK
        for j in range(3):
            pltpu.async_copy(i_hbm[j].at[pl.ds(base, _CHUNK)],
                             idx_v.at[s, j], sem_i[s])

    def wait_idx(c, s):
        base = w0 + c * _CHUNK
        for j in range(3):
            pltpu.make_async_copy(i_hbm[j].at[pl.ds(base, _CHUNK)],
                                  idx_v.at[s, j], sem_i[s]).wait()

    def start_gathers(s):
        for j in range(3):
            pltpu.async_copy(feat_hbm.at[idx_v.at[s, j]], g_v.at[s, j],
                             sem_g[s])

    def wait_gathers(s):
        for j in range(3):
            pltpu.make_async_copy(feat_hbm.at[idx_v.at[s, j]], g_v.at[s, j],
                                  sem_g[s]).wait()

    def issue_writes(c, s):
        base = w0 + c * _CHUNK
        rows = out_hbm.at[pl.ds(base, _CHUNK)]
        for j in range(3):
            pltpu.async_copy(g_v.at[s, j], rows.at[:, pl.ds(j * _D, _D)],
                             sem_w[s])

    def wait_writes(c, s):
        base = w0 + c * _CHUNK
        rows = out_hbm.at[pl.ds(base, _CHUNK)]
        for j in range(3):
            pltpu.make_async_copy(g_v.at[s, j], rows.at[:, pl.ds(j * _D, _D)],
                                  sem_w[s]).wait()

    stage_idx(0, 0)
    wait_idx(0, 0)
    start_gathers(0)
    stage_idx(1, 1)

    def pair_body(k, carry):
        for b in range(2):
            cur, nxt = b, 1 - b
            c = 2 * k + b
            if b == 0:
                @pl.when(k > 0)
                def _():
                    wait_writes(c - 1, nxt)
            else:
                wait_writes(c - 1, nxt)
            wait_idx(c + 1, nxt)
            start_gathers(nxt)
            wait_gathers(cur)
            if b == 0:
                stage_idx(c + 2, cur)
            else:
                @pl.when(k < _N_PAIRS - 1)
                def _():
                    stage_idx(c + 2, cur)
            issue_writes(c, cur)
        return carry

    lax.fori_loop(0, _N_PAIRS, pair_body, 0)

    last = _N_CHUNKS - 1
    wait_gathers(0)
    issue_writes(last, 0)
    wait_writes(last - 1, 1)
    wait_writes(last, 0)


def _sc_gather(i0, i1, i2, FEATURE):
    mesh = plsc.VectorSubcoreMesh(
        core_axis_name="c", subcore_axis_name="s",
        num_cores=_NC, num_subcores=_NS)
    f = pl.kernel(
        _sc_kernel,
        out_type=jax.ShapeDtypeStruct((_E, _GW), jnp.float32),
        mesh=mesh,
        scratch_types=[
            pltpu.VMEM((2, 3, _CHUNK), jnp.int32),
            pltpu.VMEM((2, 3, _CHUNK, _D), jnp.float32),
            pltpu.SemaphoreType.DMA,
            pltpu.SemaphoreType.DMA,
            pltpu.SemaphoreType.DMA,
            pltpu.SemaphoreType.DMA,
            pltpu.SemaphoreType.DMA,
            pltpu.SemaphoreType.DMA,
        ],
    )
    return f(i0, i1, i2, FEATURE)


_TC_BE = 2560
_RBF_W = 3 * _BINS


def _tc_assemble_kernel(d_ref, g_ref, out_ref):
    i = pl.program_id(0)
    out_ref[:, :_GW] = g_ref[:, :]
    r = lax.broadcasted_iota(jnp.int32, (1, _RBF_W), 1)
    cen = (r % _BINS).astype(jnp.float32) * ((_VMAX - _VMIN) / (_BINS - 1))
    band = r // _BINS
    gam = jnp.where(band == 0, _GAMMAS[0],
                    jnp.where(band == 1, _GAMMAS[1], _GAMMAS[2]))
    d = d_ref[pl.ds(i * _TC_BE, _TC_BE)].reshape(_TC_BE, 1)
    t = d - cen
    out_ref[:, _GW:] = jnp.exp(t * t * (-gam))


def _tc_assemble(d, g):
    return pl.pallas_call(
        _tc_assemble_kernel,
        out_shape=jax.ShapeDtypeStruct((_E, _W), jnp.float32),
        grid=(_E // _TC_BE,),
        in_specs=[
            pl.BlockSpec((_E,), lambda i: (0,)),
            pl.BlockSpec((_TC_BE, _GW), lambda i: (i, 0)),
        ],
        out_specs=pl.BlockSpec((_TC_BE, _W), lambda i: (i, 0)),
    )(d, g)


@jax.jit
def _rbf_triangle(distance, FEATURE):
    idx = distance[:, :3].astype(jnp.int32)
    d = distance[:, 3]
    g = _sc_gather(idx[:, 0], idx[:, 1], idx[:, 2], FEATURE)
    return _tc_assemble(d, g)


def kernel(distance, FEATURE):
    return _rbf_triangle(distance, FEATURE)


# speedup vs baseline: 3.0900x; 1.0446x over previous
import functools

import jax
import jax.numpy as jnp
import numpy as np
from jax import lax
from jax.experimental import pallas as pl
from jax.experimental.pallas import tpu as pltpu
from jax.experimental.pallas import tpu_sc as plsc

_VMIN, _VMAX, _BINS = 0.0, 8.0, 64
_GAMMAS = (100.0, 10.0, 1.0)
_D = 128
_E = 320000
_GW = 3 * _D
_RBF_W = 3 * _BINS
_W = _GW + _RBF_W

_NC, _NS, _L = 2, 16, 16
_NW = _NC * _NS
_PER_W = _E // _NW
_CHUNK = 80
_N_CHUNKS = _PER_W // _CHUNK
_N_PAIRS = (_N_CHUNKS - 1) // 2


_TC_BE = 2560


def _tc_rbf_kernel(d_ref, out_ref):
    i = pl.program_id(0)
    r = lax.broadcasted_iota(jnp.int32, (1, _RBF_W), 1)
    cen = (r % _BINS).astype(jnp.float32) * ((_VMAX - _VMIN) / (_BINS - 1))
    band = r // _BINS
    gam = jnp.where(band == 0, _GAMMAS[0],
                    jnp.where(band == 1, _GAMMAS[1], _GAMMAS[2]))
    d = d_ref[pl.ds(i * _TC_BE, _TC_BE)].reshape(_TC_BE, 1)
    t = d - cen
    out_ref[:, :] = jnp.exp(t * t * (-gam))


def _tc_rbf(d):
    return pl.pallas_call(
        _tc_rbf_kernel,
        out_shape=jax.ShapeDtypeStruct((_E, _RBF_W), jnp.float32),
        grid=(_E // _TC_BE,),
        in_specs=[
            pl.BlockSpec((_E,), lambda i: (0,)),
        ],
        out_specs=pl.BlockSpec((_TC_BE, _RBF_W), lambda i: (i, 0)),
    )(d)


def _sc_kernel(i0_hbm, i1_hbm, i2_hbm, rbf_hbm, feat_hbm, out_hbm,
               idx_v, g_v, rbf_v, si0, si1, sr0, sr1, sg0, sg1, sw0, sw1):
    sem_i = (si0, si1)
    sem_r = (sr0, sr1)
    sem_g = (sg0, sg1)
    sem_w = (sw0, sw1)
    i_hbm = (i0_hbm, i1_hbm, i2_hbm)
    wid = lax.axis_index("s") * _NC + lax.axis_index("c")
    w0 = wid * _PER_W

    def stage_idx(c, s):
        base = w0 + c * _CHUNK
        for j in range(3):
            pltpu.async_copy(i_hbm[j].at[pl.ds(base, _CHUNK)],
                             idx_v.at[s, j], sem_i[s])

    def wait_idx(c, s):
        base = w0 + c * _CHUNK
        for j in range(3):
            pltpu.make_async_copy(i_hbm[j].at[pl.ds(base, _CHUNK)],
                                  idx_v.at[s, j], sem_i[s]).wait()

    def stage_rbf(c, s):
        base = w0 + c * _CHUNK
        pltpu.async_copy(rbf_hbm.at[pl.ds(base, _CHUNK)], rbf_v.at[s],
                         sem_r[s])

    def wait_rbf(c, s):
        base = w0 + c * _CHUNK
        pltpu.make_async_copy(rbf_hbm.at[pl.ds(base, _CHUNK)], rbf_v.at[s],
                              sem_r[s]).wait()

    def start_gathers(s):
        for j in range(3):
            pltpu.async_copy(feat_hbm.at[idx_v.at[s, j]], g_v.at[s, j],
                             sem_g[s])

    def wait_gathers(s):
        for j in range(3):
            pltpu.make_async_copy(feat_hbm.at[idx_v.at[s, j]], g_v.at[s, j],
                                  sem_g[s]).wait()

    def issue_writes(c, s):
        base = w0 + c * _CHUNK
        rows = out_hbm.at[pl.ds(base, _CHUNK)]
        for j in range(3):
            pltpu.async_copy(g_v.at[s, j], rows.at[:, pl.ds(j * _D, _D)],
                             sem_w[s])
        pltpu.async_copy(rbf_v.at[s, :, pl.ds(0, _D)],
                         rows.at[:, pl.ds(_GW, _D)], sem_w[s])
        pltpu.async_copy(rbf_v.at[s, :, pl.ds(_D, _RBF_W - _D)],
                         rows.at[:, pl.ds(_GW + _D, _RBF_W - _D)], sem_w[s])

    def wait_writes(c, s):
        base = w0 + c * _CHUNK
        rows = out_hbm.at[pl.ds(base, _CHUNK)]
        for j in range(3):
            pltpu.make_async_copy(g_v.at[s, j], rows.at[:, pl.ds(j * _D, _D)],
                                  sem_w[s]).wait()
        pltpu.make_async_copy(rbf_v.at[s, :, pl.ds(0, _D)],
                              rows.at[:, pl.ds(_GW, _D)], sem_w[s]).wait()
        pltpu.make_async_copy(rbf_v.at[s, :, pl.ds(_D, _RBF_W - _D)],
                              rows.at[:, pl.ds(_GW + _D, _RBF_W - _D)],
                              sem_w[s]).wait()

    stage_idx(0, 0)
    stage_rbf(0, 0)
    wait_idx(0, 0)
    start_gathers(0)
    stage_idx(1, 1)
    stage_rbf(1, 1)

    def pair_body(k, carry):
        for b in range(2):
            cur, nxt = b, 1 - b
            c = 2 * k + b
            if b == 0:
                @pl.when(k > 0)
                def _():
                    wait_writes(c - 1, nxt)
                    stage_rbf(c + 1, nxt)
            else:
                wait_writes(c - 1, nxt)
                stage_rbf(c + 1, nxt)
            wait_idx(c + 1, nxt)
            start_gathers(nxt)
            wait_gathers(cur)
            if b == 0:
                stage_idx(c + 2, cur)
            else:
                @pl.when(k < _N_PAIRS - 1)
                def _():
                    stage_idx(c + 2, cur)
            wait_rbf(c, cur)
            issue_writes(c, cur)
        return carry

    lax.fori_loop(0, _N_PAIRS, pair_body, 0)

    last = _N_CHUNKS - 1
    wait_gathers(0)
    wait_rbf(last, 0)
    issue_writes(last, 0)
    wait_writes(last - 1, 1)
    wait_writes(last, 0)


def _sc_assemble(i0, i1, i2, rbf, FEATURE):
    mesh = plsc.VectorSubcoreMesh(
        core_axis_name="c", subcore_axis_name="s",
        num_cores=_NC, num_subcores=_NS)
    f = pl.kernel(
        _sc_kernel,
        out_type=jax.ShapeDtypeStruct((_E, _W), jnp.float32),
        mesh=mesh,
        scratch_types=[
            pltpu.VMEM((2, 3, _CHUNK), jnp.int32),
            pltpu.VMEM((2, 3, _CHUNK, _D), jnp.float32),
            pltpu.VMEM((2, _CHUNK, _RBF_W), jnp.float32),
            pltpu.SemaphoreType.DMA,
            pltpu.SemaphoreType.DMA,
            pltpu.SemaphoreType.DMA,
            pltpu.SemaphoreType.DMA,
            pltpu.SemaphoreType.DMA,
            pltpu.SemaphoreType.DMA,
            pltpu.SemaphoreType.DMA,
            pltpu.SemaphoreType.DMA,
        ],
    )
    return f(i0, i1, i2, rbf, FEATURE)


@jax.jit
def _rbf_triangle(distance, FEATURE):
    idx = distance[:, :3].astype(jnp.int32)
    d = distance[:, 3]
    rbf = _tc_rbf(d)
    return _sc_assemble(idx[:, 0], idx[:, 1], idx[:, 2], rbf, FEATURE)


def kernel(distance, FEATURE):
    return _rbf_triangle(distance, FEATURE)
